# baseline (speedup 1.0000x reference)
import functools

import jax
import jax.numpy as jnp
from jax import lax
from jax.experimental import pallas as pl
from jax.experimental.pallas import tpu as pltpu
from jax.experimental.pallas import tpu_sc as plsc

A_DIM = 512
T_DIM = 2048
STEPS = 16
N_ARR = 16
GROUPS = 32

NW = 32
ARR_ELEMS = A_DIM * T_DIM
PER_W = ARR_ELEMS // 4
CHUNK = 32768
NCH = PER_W // CHUNK
UNROLL = 8


def _sc_scan_body(a_hbm, b_hbm, va_hbm, fa_hbm, ga_hbm, vb_hbm, fb_hbm,
                  gb_hbm, buf0, buf1, vout, fout, gout, sem0, sem1):
    cid = lax.axis_index("c")
    sid = lax.axis_index("s")
    wid = sid * 2 + cid
    base = wid * PER_W
    cbase0 = (wid & 3) * PER_W
    lane = lax.iota(jnp.int32, 16)
    bufs = [buf0, buf1]
    sems = [sem0, sem1]
    srcs = [a_hbm] * NCH + [b_hbm] * NCH
    ninf = jnp.full((16,), -jnp.inf, jnp.float32)
    z16 = jnp.zeros((16,), jnp.float32)

    def start(ch):
        g = ch % NCH
        return pltpu.async_copy(
            srcs[ch].at[pl.ds(pl.multiple_of(base + g * CHUNK, 8), CHUNK)],
            bufs[ch % 2], sems[ch % 2])

    cp = start(0)
    outs = [(va_hbm, fa_hbm, ga_hbm), (vb_hbm, fb_hbm, gb_hbm)]
    for phase in range(2):
        top_v = ninf
        top_i = jnp.full((16,), 2**31 - 1, jnp.int32)
        thr = jnp.float32(-jnp.inf)
        gsums = z16
        for g in range(NCH):
            ch = phase * NCH + g
            if ch + 1 < 2 * NCH:
                nxt = start(ch + 1)
            else:
                nxt = None
            cp.wait()
            buf = bufs[ch % 2]

            def inner(i, carry, buf=buf):
                s0, s1, m0, m1 = carry
                b0 = i * (16 * UNROLL)
                for u in range(0, UNROLL, 2):
                    v0 = buf[pl.ds(b0 + u * 16, 16)]
                    v1 = buf[pl.ds(b0 + (u + 1) * 16, 16)]
                    s0 = s0 + v0 * v0
                    m0 = jnp.maximum(m0, v0)
                    s1 = s1 + v1 * v1
                    m1 = jnp.maximum(m1, v1)
                return s0, s1, m0, m1

            s0, s1, m0, m1 = lax.fori_loop(
                0, (CHUNK // 16) // UNROLL, inner, (z16, z16, ninf, ninf))
            gsums = jnp.where(lane == g, gsums + jnp.sum(s0 + s1), gsums)
            cmax = jnp.max(jnp.maximum(m0, m1))
            cbase = cbase0 + g * CHUNK

            def rescan(carry, buf=buf, cbase=cbase):
                def body(i, c):
                    tv, ti, th = c
                    v = buf[pl.ds(i * 16, 16)]
                    m = jnp.max(v)

                    def do(c2):
                        tv2, ti2, _ = c2
                        inc_i = lane + (cbase + i * 16)
                        sv, si = plsc.sort_key_val(v, inc_i, descending=True)
                        take = (sv > tv2) | ((sv == tv2) & (si < ti2))
                        nv = jnp.where(take, sv, tv2)
                        ni = jnp.where(take, si, ti2)
                        nv, ni = plsc.sort_key_val(nv, ni, descending=False)
                        return nv, ni, jnp.min(nv)

                    return lax.cond(m > th, do, lambda c2: c2, (tv, ti, th))

                return lax.fori_loop(0, CHUNK // 16, body, carry)

            top_v, top_i, thr = lax.cond(
                cmax > thr, rescan, lambda c: c, (top_v, top_i, thr))
            cp = nxt

        v_hbm, f_hbm, g_hbm = outs[phase]
        vout[...] = top_v
        fout[...] = top_i
        gout[...] = gsums
        pltpu.sync_copy(vout, v_hbm.at[wid])
        pltpu.sync_copy(fout, f_hbm.at[wid])
        pltpu.sync_copy(gout, g_hbm.at[wid])


@functools.cache
def _get_sc_scan():
  return functools.partial(
    pl.kernel,
    mesh=plsc.VectorSubcoreMesh(core_axis_name="c", subcore_axis_name="s"),
    compiler_params=pltpu.CompilerParams(needs_layout_passes=False),
    out_type=[
        jax.ShapeDtypeStruct((NW, 16), jnp.float32),
        jax.ShapeDtypeStruct((NW, 16), jnp.int32),
        jax.ShapeDtypeStruct((NW, 16), jnp.float32),
        jax.ShapeDtypeStruct((NW, 16), jnp.float32),
        jax.ShapeDtypeStruct((NW, 16), jnp.int32),
        jax.ShapeDtypeStruct((NW, 16), jnp.float32),
    ],
    scratch_types=[
        pltpu.VMEM((CHUNK,), jnp.float32),
        pltpu.VMEM((CHUNK,), jnp.float32),
        pltpu.VMEM((16,), jnp.float32),
        pltpu.VMEM((16,), jnp.int32),
        pltpu.VMEM((16,), jnp.float32),
        pltpu.SemaphoreType.DMA,
        pltpu.SemaphoreType.DMA,
    ],
  )(_sc_scan_body)


NCAND = 4 * STEPS
Q_ALL = N_ARR * NCAND
P_ALL = N_ARR * STEPS


def _merge_body(cv_ref, cvr_ref, cf_ref, cfr_ref, vals_ref, flats_ref):
    cv = cv_ref[...]
    cvr = cvr_ref[...]
    cf = cf_ref[...]
    cfr = cfr_ref[...]

    p_io = lax.broadcasted_iota(jnp.int32, (Q_ALL, 1), 0)
    q_io = lax.broadcasted_iota(jnp.int32, (1, Q_ALL), 1)
    same = (p_io >> 6) == (q_io >> 6)
    beats = same & ((cv > cvr) | ((cv == cvr) & (cf < cfr)))
    rank_row = jnp.sum(beats.astype(jnp.int32), axis=0, keepdims=True)

    e_io = lax.broadcasted_iota(jnp.int32, (P_ALL, 1), 0)
    sel2 = ((q_io >> 6) == (e_io >> 4)) & (rank_row == (e_io & 15))
    vals = jnp.sum(jnp.where(sel2, cvr, 0.0), axis=1, keepdims=True)
    flats = jnp.sum(jnp.where(sel2, cfr, 0), axis=1, keepdims=True)
    vals_ref[...] = vals
    flats_ref[...] = flats


def _order_body(rows_ref, kc_ref, kr_ref, vals16_ref, grp16_ref, gsum_ref,
                out_ref):
    rows = rows_ref[...]
    keys_col = kc_ref[...]
    keys_row = kr_ref[...]

    pp_io = lax.broadcasted_iota(jnp.int32, (P_ALL, 1), 0)
    qq_io = lax.broadcasted_iota(jnp.int32, (1, P_ALL), 1)
    same_s = (pp_io >> 4) == (qq_io >> 4)
    beats_s = same_s & ((keys_col < keys_row) |
                        ((keys_col == keys_row) & (pp_io < qq_io)))
    rank_s = jnp.sum(beats_s.astype(jnp.int32), axis=0, keepdims=True)
    perm = (same_s & (rank_s == (pp_io & 15))).astype(jnp.float32)
    sorted_rows = lax.dot_general(perm, rows, (((1,), (0,)), ((), ())),
                                  preferred_element_type=jnp.float32)

    half = P_ALL // 2
    diff = sorted_rows[:half, :] - sorted_rows[half:, :]
    mse = jnp.sum(diff * diff) / float(half * 128)

    g_io = lax.broadcasted_iota(jnp.int32, (1, GROUPS), 1)
    removed = jnp.zeros((N_ARR, GROUPS), jnp.float32)
    for k in range(STEPS):
        vk = vals16_ref[:, k:k + 1]
        gk = grp16_ref[:, k:k + 1]
        removed = removed + jnp.where(g_io == gk, vk * vk, 0.0)
    resid = jnp.maximum(gsum_ref[...] - removed, 0.0)
    norm = jnp.sqrt(resid)
    nh = N_ARR // 2
    nmean = jnp.sum(jnp.abs(norm[:nh, :] - norm[nh:, :])) / float(nh * GROUPS)
    out_ref[...] = jnp.full((1, 1), mse + nmean, jnp.float32)


def kernel(a, b, emb_table, ord_w):
    af = a.reshape(-1)
    bf = b.reshape(-1)
    cva, cfa, gsa, cvb, cfb, gsb = _get_sc_scan()(af, bf)
    cv = jnp.concatenate([cva, cvb], axis=0)
    cf = jnp.concatenate([cfa, cfb], axis=0)
    gs = jnp.concatenate([gsa, gsb], axis=0)

    vals, flats = pl.pallas_call(
        _merge_body,
        out_shape=[
            jax.ShapeDtypeStruct((P_ALL, 1), jnp.float32),
            jax.ShapeDtypeStruct((P_ALL, 1), jnp.int32),
        ],
    )(
        cv.reshape(Q_ALL, 1),
        cv.reshape(1, Q_ALL),
        cf.reshape(Q_ALL, 1),
        cf.reshape(1, Q_ALL),
    )
    gsum = gs.reshape(N_ARR, 4, 16)[:, :, :8].reshape(N_ARR, GROUPS)

    v = vals[:, 0]
    fl = flats[:, 0]
    atom = fl >> 11
    t = fl & (T_DIM - 1)
    pos_idx = jnp.where(v > 0, t,
                        jnp.where(v == 0, 0, jnp.where(t != 0, 0, 1)))
    atom_idx = jnp.where(v > 0, atom,
                         jnp.where(v == 0, 0, jnp.where(atom != 0, 0, 1)))
    rng = jnp.linspace(0.0, 1.0, T_DIM)
    pos = rng[pos_idx] * 20.0
    onehot = (atom_idx[:, None] ==
              jnp.arange(A_DIM)[None, :]).astype(jnp.float32)
    emb_rows = onehot @ emb_table
    rows = jnp.concatenate(
        [pos[:, None], vals, emb_rows], axis=1)
    keys = (rows.reshape(N_ARR, STEPS, 128) @ ord_w).reshape(P_ALL)

    out = pl.pallas_call(
        _order_body,
        out_shape=jax.ShapeDtypeStruct((1, 1), jnp.float32),
    )(rows, keys.reshape(P_ALL, 1), keys.reshape(1, P_ALL),
      vals.reshape(N_ARR, STEPS), (flats >> 15).reshape(N_ARR, STEPS), gsum)
    return out[0, 0]
